# baseline (speedup 1.0000x reference)
import jax
import jax.numpy as jnp
from jax import lax
from jax.experimental import pallas as pl
from jax.experimental.pallas import tpu as pltpu
from jax.experimental.pallas import tpu_sc as plsc

N_FIELDS = 26
F_DIM = 100000
B_TOTAL = 16384
ROWS = N_FIELDS * F_DIM

_info = plsc.get_sparse_core_info()
NC, NS, L = _info.num_cores, _info.num_subcores, _info.num_lanes
NW = NC * NS
B_W = B_TOTAL // NW
IDX_W = B_W * N_FIELDS
N_DMA_ROWS = IDX_W // 128


def _wid():
    return lax.axis_index("s") * NC + lax.axis_index("c")


def _idx_body(x_hbm, idx_hbm, x_v, idx_v):
    wid = _wid()

    pltpu.sync_copy(x_hbm.at[pl.ds(wid * B_W, B_W)], x_v)

    lanes = lax.iota(jnp.int32, L)
    zeros = jnp.zeros((L,), jnp.int32)

    def trans_f(f, _):
        f_vec = zeros + f
        for t in range(NW):
            b_vec = t * L + lanes
            vals = plsc.load_gather(x_v, [b_vec, f_vec]) + f * F_DIM
            idx_v.at[f * 4 + (t // 8)][pl.ds((t % 8) * L, L)] = vals
        return 0

    lax.fori_loop(0, N_FIELDS, trans_f, 0)

    pltpu.sync_copy(idx_v, idx_hbm.at[pl.ds(wid * N_DMA_ROWS, N_DMA_ROWS)])


def _gather_body(table_hbm, idx_hbm, out_hbm, idx_v, rows_v, out_v, sem):
    wid = _wid()

    pltpu.sync_copy(idx_hbm.at[pl.ds(wid * N_DMA_ROWS, N_DMA_ROWS)], idx_v)

    def fire(j, _):
        pltpu.make_async_copy(
            table_hbm.at[idx_v.at[j]], rows_v.at[j], sem
        ).start()
        return 0

    lax.fori_loop(0, N_DMA_ROWS, fire, 0)

    def drain(j, _):
        pltpu.make_async_copy(
            table_hbm.at[idx_v.at[j]], rows_v.at[j], sem
        ).wait()
        return 0

    lax.fori_loop(0, N_DMA_ROWS, drain, 0)

    lanes = lax.iota(jnp.int32, L)
    zeros = jnp.zeros((L,), jnp.int32)

    def reduce_t(t, _):
        g = t >> 3
        col_vec = (t & 7) * L + lanes
        acc = jnp.zeros((L,), jnp.float32)
        for f in range(N_FIELDS):
            acc = acc + plsc.load_gather(
                rows_v, [zeros + (f * 4 + g), col_vec, zeros]
            )
        out_v[pl.ds(t * L, L)] = acc
        return 0

    lax.fori_loop(0, NW, reduce_t, 0)

    pltpu.sync_copy(out_v, out_hbm.at[pl.ds(wid * B_W, B_W)])


@jax.jit
def _features_linear(x, table):
    mesh = plsc.VectorSubcoreMesh(core_axis_name="c", subcore_axis_name="s")
    params = pltpu.CompilerParams(needs_layout_passes=False)

    idx_all = pl.kernel(
        _idx_body,
        mesh=mesh,
        compiler_params=params,
        out_type=jax.ShapeDtypeStruct((NW * N_DMA_ROWS, 128), jnp.int32),
        scratch_types=[
            pltpu.VMEM((B_W, N_FIELDS), jnp.int32),
            pltpu.VMEM((N_DMA_ROWS, 128), jnp.int32),
        ],
    )(x)

    return pl.kernel(
        _gather_body,
        mesh=mesh,
        compiler_params=pltpu.CompilerParams(
            needs_layout_passes=False, use_tc_tiling_on_sc=False
        ),
        out_type=jax.ShapeDtypeStruct((B_TOTAL,), jnp.float32),
        scratch_types=[
            pltpu.VMEM((N_DMA_ROWS, 128), jnp.int32),
            pltpu.VMEM((N_DMA_ROWS, 128, 1), jnp.float32),
            pltpu.VMEM((B_W,), jnp.float32),
            pltpu.SemaphoreType.DMA,
        ],
    )(table, idx_all)


def kernel(x, table, bias):
    out = _features_linear(x, table)
    return out.reshape(B_TOTAL, 1) + bias


# speedup vs baseline: 27.1063x; 27.1063x over previous
import jax
import jax.numpy as jnp
from jax import lax
from jax.experimental import pallas as pl
from jax.experimental.pallas import tpu as pltpu
from jax.experimental.pallas import tpu_sc as plsc

N_FIELDS = 26
F_DIM = 100000
B_TOTAL = 16384
ROWS = N_FIELDS * F_DIM

_info = plsc.get_sparse_core_info()
NC, NS, L = _info.num_cores, _info.num_subcores, _info.num_lanes
NW = NC * NS
B_W = B_TOTAL // NW
IDX_W = B_W * N_FIELDS
N_DMA_ROWS = IDX_W // 128


def _wid():
    return lax.axis_index("s") * NC + lax.axis_index("c")


def _idx_body(x_hbm, idx_hbm, x_v, idx_v):
    wid = _wid()

    pltpu.sync_copy(x_hbm.at[pl.ds(wid * B_W, B_W)], x_v)

    lanes = lax.iota(jnp.int32, L)
    zeros = jnp.zeros((L,), jnp.int32)

    def trans_f(f, _):
        f_vec = zeros + f
        for t in range(NW):
            b_vec = t * L + lanes
            vals = plsc.load_gather(x_v, [b_vec, f_vec]) + f * F_DIM
            idx_v.at[f * 4 + (t // 8)][pl.ds((t % 8) * L, L)] = vals
        return 0

    lax.fori_loop(0, N_FIELDS, trans_f, 0)

    pltpu.sync_copy(idx_v, idx_hbm.at[pl.ds(wid * N_DMA_ROWS, N_DMA_ROWS)])


def _gather_body(table_hbm, idx_hbm, out_hbm, idx_v, rows_v, out_v, sem):
    wid = _wid()

    pltpu.sync_copy(idx_hbm.at[pl.ds(wid * N_DMA_ROWS, N_DMA_ROWS)], idx_v)

    def fire(j, _):
        pltpu.make_async_copy(
            table_hbm.at[idx_v.at[j]], rows_v.at[j], sem
        ).start()
        return 0

    lax.fori_loop(0, N_DMA_ROWS, fire, 0)

    def drain(j, _):
        pltpu.make_async_copy(
            table_hbm.at[idx_v.at[j]], rows_v.at[j], sem
        ).wait()
        return 0

    lax.fori_loop(0, N_DMA_ROWS, drain, 0)

    def reduce_t(t, _):
        g = t >> 3
        acc = jnp.zeros((L,), jnp.float32)
        for f in range(N_FIELDS):
            acc = acc + rows_v.at[f * 4 + g][pl.ds((t & 7) * L, L)]
        out_v[pl.ds(t * L, L)] = acc
        return 0

    lax.fori_loop(0, NW, reduce_t, 0)

    pltpu.sync_copy(out_v, out_hbm.at[pl.ds(wid * B_W, B_W)])


@jax.jit
def _features_linear(x, table):
    mesh = plsc.VectorSubcoreMesh(core_axis_name="c", subcore_axis_name="s")
    params = pltpu.CompilerParams(needs_layout_passes=False)

    idx_all = pl.kernel(
        _idx_body,
        mesh=mesh,
        compiler_params=params,
        out_type=jax.ShapeDtypeStruct((NW * N_DMA_ROWS, 128), jnp.int32),
        scratch_types=[
            pltpu.VMEM((B_W, N_FIELDS), jnp.int32),
            pltpu.VMEM((N_DMA_ROWS, 128), jnp.int32),
        ],
    )(x)

    table_flat = jnp.transpose(table, (1, 0)).reshape(-1)

    return pl.kernel(
        _gather_body,
        mesh=mesh,
        compiler_params=params,
        out_type=jax.ShapeDtypeStruct((B_TOTAL,), jnp.float32),
        scratch_types=[
            pltpu.VMEM((N_DMA_ROWS, 128), jnp.int32),
            pltpu.VMEM((N_DMA_ROWS, 128), jnp.float32),
            pltpu.VMEM((B_W,), jnp.float32),
            pltpu.SemaphoreType.DMA,
        ],
    )(table_flat, idx_all)


def kernel(x, table, bias):
    out = _features_linear(x, table)
    return out.reshape(B_TOTAL, 1) + bias


# speedup vs baseline: 28.1876x; 1.0399x over previous
import jax
import jax.numpy as jnp
from jax import lax
from jax.experimental import pallas as pl
from jax.experimental.pallas import tpu as pltpu
from jax.experimental.pallas import tpu_sc as plsc

N_FIELDS = 26
F_DIM = 100000
B_TOTAL = 16384
ROWS = N_FIELDS * F_DIM

_info = plsc.get_sparse_core_info()
NC, NS, L = _info.num_cores, _info.num_subcores, _info.num_lanes
NW = NC * NS
B_W = B_TOTAL // NW
IDX_W = B_W * N_FIELDS
N_DMA_ROWS = IDX_W // 128


def _wid():
    return lax.axis_index("s") * NC + lax.axis_index("c")


def _idx_body(xt_hbm, idx_hbm, x_v, idx_v):
    wid = _wid()

    pltpu.sync_copy(xt_hbm.at[:, pl.ds(wid * B_W, B_W)], x_v)

    lanes = lax.iota(jnp.int32, L)
    zeros = jnp.zeros((L,), jnp.int32)

    def off_f(f, _):
        f_vec = zeros + f
        for t in range(NW):
            vals = plsc.load_gather(x_v, [f_vec, t * L + lanes]) + f * F_DIM
            idx_v.at[f * 4 + (t // 8)][pl.ds((t % 8) * L, L)] = vals
        return 0

    lax.fori_loop(0, N_FIELDS, off_f, 0)

    pltpu.sync_copy(idx_v, idx_hbm.at[pl.ds(wid * N_DMA_ROWS, N_DMA_ROWS)])


def _gather_body(table_hbm, idx_hbm, out_hbm, idx_v, rows_v, out_v, sem):
    wid = _wid()

    pltpu.sync_copy(idx_hbm.at[pl.ds(wid * N_DMA_ROWS, N_DMA_ROWS)], idx_v)

    def fire(j, _):
        pltpu.make_async_copy(
            table_hbm.at[idx_v.at[j]], rows_v.at[j], sem
        ).start()
        return 0

    lax.fori_loop(0, N_DMA_ROWS, fire, 0)

    def drain(j, _):
        pltpu.make_async_copy(
            table_hbm.at[idx_v.at[j]], rows_v.at[j], sem
        ).wait()
        return 0

    lax.fori_loop(0, N_DMA_ROWS, drain, 0)

    def reduce_t(t, _):
        g = t >> 3
        acc = jnp.zeros((L,), jnp.float32)
        for f in range(N_FIELDS):
            acc = acc + rows_v.at[f * 4 + g][pl.ds((t & 7) * L, L)]
        out_v[pl.ds(t * L, L)] = acc
        return 0

    lax.fori_loop(0, NW, reduce_t, 0)

    pltpu.sync_copy(out_v, out_hbm.at[pl.ds(wid * B_W, B_W)])


@jax.jit
def _features_linear(x, table):
    mesh = plsc.VectorSubcoreMesh(core_axis_name="c", subcore_axis_name="s")
    params = pltpu.CompilerParams(needs_layout_passes=False)

    idx_all = pl.kernel(
        _idx_body,
        mesh=mesh,
        compiler_params=params,
        out_type=jax.ShapeDtypeStruct((NW * N_DMA_ROWS, 128), jnp.int32),
        scratch_types=[
            pltpu.VMEM((N_FIELDS, B_W), jnp.int32),
            pltpu.VMEM((N_DMA_ROWS, 128), jnp.int32),
        ],
    )(jnp.transpose(x, (1, 0)))

    table_flat = table[:, 0]

    return pl.kernel(
        _gather_body,
        mesh=mesh,
        compiler_params=params,
        out_type=jax.ShapeDtypeStruct((B_TOTAL,), jnp.float32),
        scratch_types=[
            pltpu.VMEM((N_DMA_ROWS, 128), jnp.int32),
            pltpu.VMEM((N_DMA_ROWS, 128), jnp.float32),
            pltpu.VMEM((B_W,), jnp.float32),
            pltpu.SemaphoreType.DMA,
        ],
    )(table_flat, idx_all)


def kernel(x, table, bias):
    out = _features_linear(x, table)
    return out.reshape(B_TOTAL, 1) + bias
